# baseline (speedup 1.0000x reference)
import functools

import jax
import jax.numpy as jnp
from jax import lax
from jax.experimental import pallas as pl
from jax.experimental.pallas import tpu as pltpu
from jax.experimental.pallas import tpu_sc as plsc

_VOCAB = 100000
_DIM = 128
_NCLS = 3
_BATCH = 4096
_SEQ = 200
_PAD = 16

_NC = 2
_NS = 16
_NW = _NC * _NS
_BPW = _BATCH // _NW
_TPW = _BPW * _SEQ
_GRP = 2
_CHUNK = _GRP * _SEQ
_NITER = _TPW // _CHUNK
_SPLITS = ((0, 128), (128, 128), (256, 128), (384, 16))
_UNROLL = 8


def _proj_body(t_ref, w_ref, b_ref, o_ref):
    o_ref[...] = (
        jnp.dot(t_ref[...], w_ref[...], preferred_element_type=jnp.float32)
        + b_ref[...]
    )


def _project(table, w_s, b_s):
    blk = 10000
    return pl.pallas_call(
        _proj_body,
        grid=(_VOCAB // blk,),
        in_specs=[
            pl.BlockSpec((blk, _DIM), lambda i: (i, 0)),
            pl.BlockSpec((_DIM, _PAD), lambda i: (0, 0)),
            pl.BlockSpec((1, _PAD), lambda i: (0, 0)),
        ],
        out_specs=pl.BlockSpec((blk, _PAD), lambda i: (i, 0)),
        out_shape=jax.ShapeDtypeStruct((_VOCAB, _PAD), jnp.float32),
    )(table, w_s, b_s)


_mesh = plsc.VectorSubcoreMesh(core_axis_name="c", subcore_axis_name="s")

_SUNROLL = 8


_TCHUNKS = tuple(range(0, _SEQ - 16, 16)) + (_SEQ - 16,)


@functools.partial(
    pl.kernel,
    mesh=_mesh,
    compiler_params=pltpu.CompilerParams(
        use_tc_tiling_on_sc=False, needs_layout_passes=False),
    out_type=jax.ShapeDtypeStruct((_BATCH, _PAD), jnp.float32),
    scratch_types=[
        pltpu.VMEM((_BPW, _SEQ), jnp.int32),
        pltpu.VMEM((_SEQ, _BPW), jnp.int32),
        pltpu.VMEM((_BPW, _PAD), jnp.float32),
        pltpu.SemaphoreType.DMA,
        pltpu.SemaphoreType.DMA,
    ],
)
def _sc_pool(text_hbm, p_hbm, out_hbm, loc, slab, acc, tsem, gsem):
    wid = lax.axis_index("s") * _NC + lax.axis_index("c")
    b0 = wid * _BPW

    pltpu.make_async_copy(
        text_hbm.at[pl.ds(b0, _BPW), :], loc, tsem
    ).start()

    zero = jnp.zeros((_PAD,), jnp.float32)
    def zbody(i, carry):
        acc[i, :] = zero
        return carry
    lax.fori_loop(0, _BPW, zbody, 0)

    pltpu.make_async_copy(
        text_hbm.at[pl.ds(b0, _BPW), :], loc, tsem
    ).wait()

    base = lax.iota(jnp.int32, 16)
    rowidx = tuple(base + off for off in _TCHUNKS)

    def tbody(bl, carry):
        colidx = jnp.full((16,), bl, jnp.int32)
        for ci, off in enumerate(_TCHUNKS):
            v = loc[bl, pl.ds(off, 16)]
            plsc.store_scatter(slab, [rowidx[ci], colidx], v)
        return carry
    lax.fori_loop(0, _BPW, tbody, 0)

    def fire(j, carry):
        for u in range(_SUNROLL):
            pltpu.make_async_copy(
                p_hbm.at[slab.at[j * _SUNROLL + u]], acc, gsem
            ).start(add=True)
        return carry
    lax.fori_loop(0, _SEQ // _SUNROLL, fire, 0)

    def drain(j, carry):
        for u in range(_SUNROLL):
            pltpu.make_async_copy(
                p_hbm.at[slab.at[j * _SUNROLL + u]], acc, gsem
            ).wait()
        return carry
    lax.fori_loop(0, _SEQ // _SUNROLL, drain, 0)

    pltpu.sync_copy(acc, out_hbm.at[pl.ds(b0, _BPW)])


def kernel(text, table, W, b):
    inv = jnp.float32(1.0 / _SEQ)
    w_s = jnp.pad(W, ((0, 0), (0, _PAD - _NCLS))) * inv
    b_s = (jnp.pad(b, (0, _PAD - _NCLS)) * inv).reshape(1, _PAD)
    p = _project(table, w_s, b_s)
    out = _sc_pool(text.astype(jnp.int32), p)
    return out[:, :_NCLS]


# speedup vs baseline: 1.1365x; 1.1365x over previous
import functools

import jax
import jax.numpy as jnp
from jax import lax
from jax.experimental import pallas as pl
from jax.experimental.pallas import tpu as pltpu
from jax.experimental.pallas import tpu_sc as plsc

_VOCAB = 100000
_DIM = 128
_NCLS = 3
_BATCH = 4096
_SEQ = 200
_PAD = 16

_NC = 2
_NS = 16
_NW = _NC * _NS
_BPW = _BATCH // _NW
_TPW = _BPW * _SEQ
_GRP = 2
_CHUNK = _GRP * _SEQ
_NITER = _TPW // _CHUNK
_SPLITS = ((0, 128), (128, 128), (256, 128), (384, 16))
_UNROLL = 8


def _proj_body(t_ref, w_ref, b_ref, o_ref):
    o_ref[...] = (
        jnp.dot(t_ref[...], w_ref[...], preferred_element_type=jnp.float32)
        + b_ref[...]
    )


def _project(table8, w_bd, b_bd):
    blk = 1600
    return pl.pallas_call(
        _proj_body,
        grid=(pl.cdiv(_VOCAB // 8, blk),),
        in_specs=[
            pl.BlockSpec((blk, 8 * _DIM), lambda i: (i, 0)),
            pl.BlockSpec((8 * _DIM, 8 * _PAD), lambda i: (0, 0)),
            pl.BlockSpec((1, 8 * _PAD), lambda i: (0, 0)),
        ],
        out_specs=pl.BlockSpec((blk, 8 * _PAD), lambda i: (i, 0)),
        out_shape=jax.ShapeDtypeStruct((_VOCAB // 8, 8 * _PAD), jnp.float32),
    )(table8, w_bd, b_bd)


_mesh = plsc.VectorSubcoreMesh(core_axis_name="c", subcore_axis_name="s")

_SUNROLL = 8


@functools.partial(
    pl.kernel,
    mesh=_mesh,
    compiler_params=pltpu.CompilerParams(use_tc_tiling_on_sc=False),
    out_type=jax.ShapeDtypeStruct((_BATCH, _PAD), jnp.float32),
    scratch_types=[
        pltpu.VMEM((_SEQ, _BPW), jnp.int32),
        pltpu.VMEM((_BPW, _PAD), jnp.float32),
        pltpu.SemaphoreType.DMA,
        pltpu.SemaphoreType.DMA,
    ],
)
def _sc_pool(textT_hbm, p_hbm, out_hbm, slab, acc, tsem, gsem):
    wid = lax.axis_index("s") * _NC + lax.axis_index("c")
    b0 = wid * _BPW

    pltpu.make_async_copy(
        textT_hbm.at[:, pl.ds(b0, _BPW)], slab, tsem
    ).start()

    zero = jnp.zeros((_PAD,), jnp.float32)
    def zbody(i, carry):
        acc[i, :] = zero
        return carry
    lax.fori_loop(0, _BPW, zbody, 0)

    pltpu.make_async_copy(
        textT_hbm.at[:, pl.ds(b0, _BPW)], slab, tsem
    ).wait()

    def fire(j, carry):
        for u in range(_SUNROLL):
            pltpu.make_async_copy(
                p_hbm.at[slab.at[j * _SUNROLL + u]], acc, gsem
            ).start(add=True)
        return carry
    lax.fori_loop(0, _SEQ // _SUNROLL, fire, 0)

    def drain(j, carry):
        for u in range(_SUNROLL):
            pltpu.make_async_copy(
                p_hbm.at[slab.at[j * _SUNROLL + u]], acc, gsem
            ).wait()
        return carry
    lax.fori_loop(0, _SEQ // _SUNROLL, drain, 0)

    pltpu.sync_copy(acc, out_hbm.at[pl.ds(b0, _BPW)])


def kernel(text, table, W, b):
    inv = jnp.float32(1.0 / _SEQ)
    w_s = jnp.pad(W, ((0, 0), (0, _PAD - _NCLS))) * inv
    b_s = jnp.pad(b, (0, _PAD - _NCLS)) * inv
    eye8 = jnp.eye(8, dtype=jnp.float32)
    w_bd = jnp.einsum("ij,kl->ikjl", eye8, w_s).reshape(8 * _DIM, 8 * _PAD)
    b_bd = jnp.tile(b_s, 8).reshape(1, 8 * _PAD)
    table8 = table.reshape(_VOCAB // 8, 8 * _DIM)
    p = _project(table8, w_bd, b_bd).reshape(_VOCAB, _PAD)
    out = _sc_pool(text.astype(jnp.int32).T, p)
    return out[:, :_NCLS]


# speedup vs baseline: 1.5912x; 1.4001x over previous
import functools

import jax
import jax.numpy as jnp
from jax import lax
from jax.experimental import pallas as pl
from jax.experimental.pallas import tpu as pltpu
from jax.experimental.pallas import tpu_sc as plsc

_VOCAB = 100000
_DIM = 128
_NCLS = 3
_BATCH = 4096
_SEQ = 200
_PAD = 16

_NC = 2
_NS = 16
_NW = _NC * _NS
_BPW = _BATCH // _NW
_TPW = _BPW * _SEQ
_GRP = 2
_CHUNK = _GRP * _SEQ
_NITER = _TPW // _CHUNK
_SPLITS = ((0, 128), (128, 128), (256, 128), (384, 16))
_UNROLL = 8


def _proj_body(t_ref, w_ref, b_ref, o_ref):
    o_ref[...] = (
        jnp.dot(t_ref[...], w_ref[...], preferred_element_type=jnp.float32)
        + b_ref[...]
    )


def _project(table, w_wide, b_wide):
    blk = 10000
    return pl.pallas_call(
        _proj_body,
        grid=(_VOCAB // blk,),
        in_specs=[
            pl.BlockSpec((blk, _DIM), lambda i: (i, 0)),
            pl.BlockSpec((_DIM, _DIM), lambda i: (0, 0)),
            pl.BlockSpec((1, _DIM), lambda i: (0, 0)),
        ],
        out_specs=pl.BlockSpec((blk, _DIM), lambda i: (i, 0)),
        out_shape=jax.ShapeDtypeStruct((_VOCAB, _DIM), jnp.float32),
    )(table, w_wide, b_wide)


_mesh = plsc.VectorSubcoreMesh(core_axis_name="c", subcore_axis_name="s")

_SUNROLL = 8


@functools.partial(
    pl.kernel,
    mesh=_mesh,
    compiler_params=pltpu.CompilerParams(use_tc_tiling_on_sc=False),
    out_type=jax.ShapeDtypeStruct((_BATCH, _PAD), jnp.float32),
    scratch_types=[
        pltpu.VMEM((_SEQ, _BPW), jnp.int32),
        pltpu.VMEM((_BPW, _PAD), jnp.float32),
        pltpu.SemaphoreType.DMA,
        pltpu.SemaphoreType.DMA,
    ],
)
def _sc_pool(textT_hbm, p_hbm, out_hbm, slab, acc, tsem, gsem):
    wid = lax.axis_index("s") * _NC + lax.axis_index("c")
    b0 = wid * _BPW

    pltpu.make_async_copy(
        textT_hbm.at[:, pl.ds(b0, _BPW)], slab, tsem
    ).start()

    zero = jnp.zeros((_PAD,), jnp.float32)
    def zbody(i, carry):
        acc[i, :] = zero
        return carry
    lax.fori_loop(0, _BPW, zbody, 0)

    pltpu.make_async_copy(
        textT_hbm.at[:, pl.ds(b0, _BPW)], slab, tsem
    ).wait()

    def fire(j, carry):
        for u in range(_SUNROLL):
            pltpu.make_async_copy(
                p_hbm.at[slab.at[j * _SUNROLL + u]], acc, gsem
            ).start(add=True)
        return carry
    lax.fori_loop(0, _SEQ // _SUNROLL, fire, 0)

    def drain(j, carry):
        for u in range(_SUNROLL):
            pltpu.make_async_copy(
                p_hbm.at[slab.at[j * _SUNROLL + u]], acc, gsem
            ).wait()
        return carry
    lax.fori_loop(0, _SEQ // _SUNROLL, drain, 0)

    pltpu.sync_copy(acc, out_hbm.at[pl.ds(b0, _BPW)])


def kernel(text, table, W, b):
    inv = jnp.float32(1.0 / _SEQ)
    w_wide = jnp.pad(W, ((0, 0), (0, _DIM - _NCLS))) * inv
    b_wide = (jnp.pad(b, (0, _DIM - _NCLS)) * inv).reshape(1, _DIM)
    p = _project(table, w_wide, b_wide).reshape(8 * _VOCAB, _PAD)
    out = _sc_pool((text.astype(jnp.int32) * 8).T, p)
    return out[:, :_NCLS]
